# baseline (speedup 1.0000x reference)
import functools

import jax
import jax.numpy as jnp
from jax import lax
from jax.experimental import pallas as pl
from jax.experimental.pallas import tpu as pltpu
from jax.experimental.pallas import tpu_sc as plsc

D_MODEL = 1024
VOCAB = 60
LANES = 16
NW = 32
DEPTH = 32
PER_W = 512
ROWS = NW * PER_W


def _build():
  mesh = plsc.VectorSubcoreMesh(core_axis_name="c", subcore_axis_name="s")

  @functools.partial(
      pl.kernel,
      mesh=mesh,
      out_type=jax.ShapeDtypeStruct((4, ROWS // 4, 1, D_MODEL), jnp.float32),
      scratch_types=[
          pltpu.VMEM((VOCAB, D_MODEL), jnp.float32),
          pltpu.VMEM((PER_W + LANES,), jnp.int32),
          pltpu.SemaphoreType.DMA,
          pltpu.SemaphoreType.DMA,
      ],
  )
  def emb(table_hbm, idx_hbm, out_hbm, tab_v, idx_v, sem, sem_in):
    wid = lax.axis_index("s") * 2 + lax.axis_index("c")
    cp_tab = pltpu.async_copy(table_hbm, tab_v, sem_in)
    cp_idx = pltpu.async_copy(idx_hbm.at[pl.ds(wid * PER_W, PER_W)],
                              idx_v.at[pl.ds(0, PER_W)], sem_in)
    cp_tab.wait()
    cp_idx.wait()

    row0 = wid * PER_W
    bidx = row0 // 4096
    t0 = row0 % 4096

    def fire(j):
      iv = idx_v[pl.ds(j, LANES)]
      v = iv[0]
      pltpu.async_copy(
          tab_v.at[pl.ds(v, 1)],
          out_hbm.at[bidx, pl.ds(t0 + j, 1), 0],
          sem)

    def wait_one():
      pltpu.make_async_copy(
          tab_v.at[pl.ds(0, 1)],
          out_hbm.at[bidx, pl.ds(t0, 1), 0],
          sem).wait()

    for j in range(DEPTH):
      fire(j)

    def body(j, _):
      wait_one()
      fire(j + DEPTH)
      return 0

    lax.fori_loop(0, PER_W - DEPTH, body, 0)
    for _ in range(DEPTH):
      wait_one()

  return emb


_emb = _build()


def kernel(x_mark, W):
  idx = x_mark[:, :, 1].reshape(ROWS).astype(jnp.int32)
  return _emb(W, idx)


# speedup vs baseline: 1.0267x; 1.0267x over previous
import functools

import jax
import jax.numpy as jnp
from jax import lax
from jax.experimental import pallas as pl
from jax.experimental.pallas import tpu as pltpu
from jax.experimental.pallas import tpu_sc as plsc

D_MODEL = 1024
VOCAB = 60
LANES = 16
NW = 32
DEPTH = 16
PER_W = 512
ROWS = NW * PER_W


def _build():
  mesh = plsc.VectorSubcoreMesh(core_axis_name="c", subcore_axis_name="s")

  @functools.partial(
      pl.kernel,
      mesh=mesh,
      out_type=jax.ShapeDtypeStruct((4, ROWS // 4, 1, D_MODEL), jnp.float32),
      scratch_types=[
          pltpu.VMEM((VOCAB, D_MODEL), jnp.float32),
          pltpu.VMEM((PER_W + LANES,), jnp.int32),
          pltpu.SemaphoreType.DMA,
          pltpu.SemaphoreType.DMA,
      ],
  )
  def emb(table_hbm, idx_hbm, out_hbm, tab_v, idx_v, sem, sem_in):
    wid = lax.axis_index("s") * 2 + lax.axis_index("c")
    cp_tab = pltpu.async_copy(table_hbm, tab_v, sem_in)
    cp_idx = pltpu.async_copy(idx_hbm.at[pl.ds(wid * PER_W, PER_W)],
                              idx_v.at[pl.ds(0, PER_W)], sem_in)
    cp_tab.wait()
    cp_idx.wait()

    row0 = wid * PER_W
    bidx = row0 // 4096
    t0 = row0 % 4096

    def fire(j):
      iv = idx_v[pl.ds(j, LANES)]
      v = iv[0]
      pltpu.async_copy(
          tab_v.at[pl.ds(v, 1)],
          out_hbm.at[bidx, pl.ds(t0 + j, 1), 0],
          sem)

    def wait_one():
      pltpu.make_async_copy(
          tab_v.at[pl.ds(0, 1)],
          out_hbm.at[bidx, pl.ds(t0, 1), 0],
          sem).wait()

    for j in range(DEPTH):
      fire(j)

    def body(j, _):
      wait_one()
      fire(j + DEPTH)
      return 0

    lax.fori_loop(0, PER_W - DEPTH, body, 0)
    for _ in range(DEPTH):
      wait_one()

  return emb


_emb = _build()


def kernel(x_mark, W):
  idx = x_mark[:, :, 1].reshape(ROWS).astype(jnp.int32)
  return _emb(W, idx)
